# baseline (speedup 1.0000x reference)
import jax
import jax.numpy as jnp
from jax import lax
from jax.experimental import pallas as pl
from jax.experimental.pallas import tpu as pltpu
from jax.experimental.pallas import tpu_sc as plsc

B, F, L, D, V = 1024, 26, 20, 128, 1000
N = B * F
NW = 32
PW = N // NW
CH = 4
ROWS = CH * L
NG = PW // CH
ZROW = F * (V + 1)
LANES = 16


def _take(vec, idx):
    dnums = lax.GatherDimensionNumbers(
        offset_dims=(), collapsed_slice_dims=(0,), start_index_map=(0,))
    return lax.gather(vec, idx[:, None], dnums, (1,),
                      mode=lax.GatherScatterMode.PROMISE_IN_BOUNDS)


def _body(ids_hbm, len_hbm, base_hbm, table_hbm, out_hbm,
          ids_v, len_v, base_v, scale_v, idx_v, rows_v, outc_v, gsem):
    c = lax.axis_index("c")
    s = lax.axis_index("s")
    wid = s * 2 + c
    pstart = wid * PW

    pltpu.sync_copy(ids_hbm.at[pl.ds(pstart * L, PW * L)], ids_v)
    pltpu.sync_copy(len_hbm.at[pl.ds(pstart, PW)], len_v.at[pl.ds(0, PW)])
    pltpu.sync_copy(base_hbm.at[pl.ds(pstart, PW)], base_v.at[pl.ds(0, PW)])

    @pl.loop(0, PW // LANES)
    def _scale(k):
        l16 = len_v[pl.ds(k * LANES, LANES)]
        scale_v[pl.ds(k * LANES, LANES)] = 1.0 / jnp.maximum(l16, 1).astype(jnp.float32)

    lane = lax.iota(jnp.int32, LANES)
    segoff = []
    posoff = []
    for j in range(ROWS // LANES):
        q = lane + (j * LANES)
        so = lax.shift_right_logical(q * 205, 12)
        segoff.append(so)
        posoff.append(q - so * L)

    @pl.loop(0, NG)
    def _index(g):
        len16c = len_v[pl.ds(g * CH, LANES)]
        base16c = base_v[pl.ds(g * CH, LANES)]
        for j in range(ROWS // LANES):
            id16 = ids_v[pl.ds(g * ROWS + j * LANES, LANES)]
            len16 = _take(len16c, segoff[j])
            b16 = _take(base16c, segoff[j])
            idx = jnp.where(posoff[j] < len16, b16 + id16, ZROW)
            idx_v[g, pl.ds(j * LANES, LANES)] = idx

    @pl.loop(0, NG)
    def _pool(g):
        pltpu.async_copy(table_hbm.at[idx_v.at[g]], rows_v, gsem).wait()
        scale16c = scale_v[pl.ds(g * CH, LANES)]
        for p in range(CH):
            sc = _take(scale16c, jnp.full((LANES,), p, jnp.int32))
            for cc in range(D // LANES):
                acc = rows_v[p * L, pl.ds(cc * LANES, LANES)]
                for l in range(1, L):
                    acc = acc + rows_v[p * L + l, pl.ds(cc * LANES, LANES)]
                outc_v[p, pl.ds(cc * LANES, LANES)] = acc * sc
        pltpu.sync_copy(outc_v, out_hbm.at[pl.ds(pstart + g * CH, CH)])


@jax.jit
def _pooled(ids_flat, lens_flat, rowbase, table_aug):
    mesh = plsc.VectorSubcoreMesh(core_axis_name="c", subcore_axis_name="s")
    return pl.kernel(
        _body,
        out_type=jax.ShapeDtypeStruct((N, D), jnp.float32),
        mesh=mesh,
        scratch_types=[
            pltpu.VMEM((PW * L,), jnp.int32),
            pltpu.VMEM((PW + LANES,), jnp.int32),
            pltpu.VMEM((PW + LANES,), jnp.int32),
            pltpu.VMEM((PW + LANES,), jnp.float32),
            pltpu.VMEM((NG, ROWS), jnp.int32),
            pltpu.VMEM((ROWS, D), jnp.float32),
            pltpu.VMEM((CH, D), jnp.float32),
            pltpu.SemaphoreType.DMA,
        ],
    )(ids_flat, lens_flat, rowbase, table_aug)


def kernel(dynamic_ids, dynamic_lengths, tables):
    ids_flat = dynamic_ids.astype(jnp.int32).reshape(N * L)
    lens_flat = dynamic_lengths.astype(jnp.int32).reshape(N)
    rowbase = jnp.tile(jnp.arange(F, dtype=jnp.int32) * (V + 1), B)
    table_aug = jnp.concatenate(
        [tables.reshape(F * (V + 1), D), jnp.zeros((1, D), jnp.float32)], axis=0)
    out = _pooled(ids_flat, lens_flat, rowbase, table_aug)
    return out.reshape(B, F, D)


# speedup vs baseline: 1.0018x; 1.0018x over previous
import jax
import jax.numpy as jnp
from jax import lax
from jax.experimental import pallas as pl
from jax.experimental.pallas import tpu as pltpu
from jax.experimental.pallas import tpu_sc as plsc

B, F, L, D, V = 1024, 26, 20, 128, 1000
N = B * F
NW = 32
PW = N // NW
CH = 4
ROWS = CH * L
NG = PW // CH
ZROW = F * (V + 1)
LANES = 16


def _take(vec, idx):
    dnums = lax.GatherDimensionNumbers(
        offset_dims=(), collapsed_slice_dims=(0,), start_index_map=(0,))
    return lax.gather(vec, idx[:, None], dnums, (1,),
                      mode=lax.GatherScatterMode.PROMISE_IN_BOUNDS)


def _body(ids_hbm, len_hbm, base_hbm, table_hbm, out_hbm,
          ids_v, len_v, base_v, scale_v, idx_v, rows_v, outc_v, gsem):
    c = lax.axis_index("c")
    s = lax.axis_index("s")
    wid = s * 2 + c
    pstart = wid * PW

    pltpu.sync_copy(ids_hbm.at[pl.ds(pstart * L, PW * L)], ids_v)
    pltpu.sync_copy(len_hbm.at[pl.ds(pstart, PW)], len_v.at[pl.ds(0, PW)])
    pltpu.sync_copy(base_hbm.at[pl.ds(pstart, PW)], base_v.at[pl.ds(0, PW)])

    @pl.loop(0, PW // LANES)
    def _scale(k):
        l16 = len_v[pl.ds(k * LANES, LANES)]
        scale_v[pl.ds(k * LANES, LANES)] = 1.0 / jnp.maximum(l16, 1).astype(jnp.float32)

    lane = lax.iota(jnp.int32, LANES)
    segoff = []
    posoff = []
    for j in range(ROWS // LANES):
        q = lane + (j * LANES)
        so = lax.shift_right_logical(q * 205, 12)
        segoff.append(so)
        posoff.append(q - so * L)

    with jax.named_scope("ph_index"):
        @pl.loop(0, NG)
        def _index(g):
            len16c = len_v[pl.ds(g * CH, LANES)]
            base16c = base_v[pl.ds(g * CH, LANES)]
            for j in range(ROWS // LANES):
                id16 = ids_v[pl.ds(g * ROWS + j * LANES, LANES)]
                len16 = _take(len16c, segoff[j])
                b16 = _take(base16c, segoff[j])
                idx = jnp.where(posoff[j] < len16, b16 + id16, ZROW)
                idx_v[g, pl.ds(j * LANES, LANES)] = idx

    with jax.named_scope("ph_pool"):
        @pl.loop(0, NG)
        def _pool(g):
            pltpu.async_copy(table_hbm.at[idx_v.at[g]], rows_v, gsem).wait()
            scale16c = scale_v[pl.ds(g * CH, LANES)]
            for p in range(CH):
                sc = _take(scale16c, jnp.full((LANES,), p, jnp.int32))
                for cc in range(D // LANES):
                    acc = rows_v[p * L, pl.ds(cc * LANES, LANES)]
                    for l in range(1, L):
                        acc = acc + rows_v[p * L + l, pl.ds(cc * LANES, LANES)]
                    outc_v[p, pl.ds(cc * LANES, LANES)] = acc * sc
            pltpu.sync_copy(outc_v, out_hbm.at[pl.ds(pstart + g * CH, CH)])


@jax.jit
def _pooled(ids_flat, lens_flat, rowbase, table_aug):
    mesh = plsc.VectorSubcoreMesh(core_axis_name="c", subcore_axis_name="s")
    return pl.kernel(
        _body,
        out_type=jax.ShapeDtypeStruct((N, D), jnp.float32),
        mesh=mesh,
        scratch_types=[
            pltpu.VMEM((PW * L,), jnp.int32),
            pltpu.VMEM((PW + LANES,), jnp.int32),
            pltpu.VMEM((PW + LANES,), jnp.int32),
            pltpu.VMEM((PW + LANES,), jnp.float32),
            pltpu.VMEM((NG, ROWS), jnp.int32),
            pltpu.VMEM((ROWS, D), jnp.float32),
            pltpu.VMEM((CH, D), jnp.float32),
            pltpu.SemaphoreType.DMA,
        ],
    )(ids_flat, lens_flat, rowbase, table_aug)


def kernel(dynamic_ids, dynamic_lengths, tables):
    ids_flat = dynamic_ids.astype(jnp.int32).reshape(N * L)
    lens_flat = dynamic_lengths.astype(jnp.int32).reshape(N)
    rowbase = jnp.tile(jnp.arange(F, dtype=jnp.int32) * (V + 1), B)
    table_aug = jnp.concatenate(
        [tables.reshape(F * (V + 1), D), jnp.zeros((1, D), jnp.float32)], axis=0)
    out = _pooled(ids_flat, lens_flat, rowbase, table_aug)
    return out.reshape(B, F, D)
